# baseline (speedup 1.0000x reference)
import functools

import jax
import jax.numpy as jnp
from jax import lax
from jax.experimental import pallas as pl
from jax.experimental.pallas import tpu as pltpu
from jax.experimental.pallas import tpu_sc as plsc

SCALING = 32.0 / 16.0
ALPHA = 1.702
LIMIT = 7.0

TM = 256
SC_CHUNK = 64


def _sc_gather_rows(table, idx):
    n_rows = idx.shape[0]
    width = table.shape[1]
    info = plsc.get_sparse_core_info()
    nc, ns = info.num_cores, info.num_subcores
    nw = nc * ns
    rows_per_w = n_rows // nw
    assert rows_per_w * nw == n_rows
    chunk = next(c for c in (64, 48, 32, 16, 8) if rows_per_w % c == 0)

    mesh = plsc.VectorSubcoreMesh(core_axis_name="c", subcore_axis_name="s")

    @functools.partial(
        pl.kernel, mesh=mesh,
        out_type=jax.ShapeDtypeStruct((n_rows, width), jnp.float32),
        scratch_types=[
            pltpu.VMEM((chunk,), jnp.int32),
            pltpu.VMEM((chunk, width), jnp.float32),
            pltpu.SemaphoreType.DMA,
        ],
    )
    def gather_k(idx_hbm, table_hbm, out_hbm, idx_v, rows_v, sem):
        wid = lax.axis_index("s") * nc + lax.axis_index("c")
        base = wid * rows_per_w
        for c in range(rows_per_w // chunk):
            off = base + c * chunk
            pltpu.sync_copy(idx_hbm.at[pl.ds(off, chunk)], idx_v)
            pltpu.async_copy(table_hbm.at[idx_v], rows_v, sem).wait()
            pltpu.sync_copy(rows_v, out_hbm.at[pl.ds(off, chunk)])

    return gather_k(idx, table)


def _sc_scatter_rows(table, idx, out_rows):
    n_rows, width = table.shape
    info = plsc.get_sparse_core_info()
    nc, ns = info.num_cores, info.num_subcores
    nw = nc * ns
    rows_per_w = n_rows // nw
    assert rows_per_w * nw == n_rows
    chunk = next(c for c in (64, 48, 32, 16, 8) if rows_per_w % c == 0)

    mesh = plsc.VectorSubcoreMesh(core_axis_name="c", subcore_axis_name="s")

    @functools.partial(
        pl.kernel, mesh=mesh,
        out_type=jax.ShapeDtypeStruct((out_rows, width), jnp.float32),
        scratch_types=[
            pltpu.VMEM((chunk,), jnp.int32),
            pltpu.VMEM((chunk, width), jnp.float32),
            pltpu.SemaphoreType.DMA,
        ],
    )
    def scatter_k(idx_hbm, table_hbm, out_hbm, idx_v, rows_v, sem):
        wid = lax.axis_index("s") * nc + lax.axis_index("c")
        base = wid * rows_per_w
        for c in range(rows_per_w // chunk):
            off = base + c * chunk
            pltpu.sync_copy(idx_hbm.at[pl.ds(off, chunk)], idx_v)
            pltpu.sync_copy(table_hbm.at[pl.ds(off, chunk)], rows_v)
            pltpu.async_copy(rows_v, out_hbm.at[idx_v], sem).wait()

    return scatter_k(idx, table)


def _gateup_body(te_ref, x_ref, wgu_ref, bgu_b_ref, agu_ref, bgu_l_ref,
                 gated_ref, gu_ref, *, num_experts):
    i = pl.program_id(0)
    f32 = jnp.float32
    bf16 = jnp.bfloat16
    inner = gated_ref.shape[0]

    @pl.when(te_ref[i] < num_experts)
    def _():
        x = x_ref[...]
        p = jnp.dot(x, agu_ref[0], preferred_element_type=f32).astype(bf16)
        gu_t = lax.dot_general(wgu_ref[0], x, (((0,), (1,)), ((), ())),
                               preferred_element_type=f32)
        dlt = lax.dot_general(bgu_l_ref[0], p, (((0,), (1,)), ((), ())),
                              preferred_element_type=f32)
        gu_t = gu_t + bgu_b_ref[0] + SCALING * dlt
        for k in range(gu_t.shape[1] // 128):
            gu_ref[...] = gu_t[:, k * 128:(k + 1) * 128]
            g = jnp.minimum(gu_ref[pl.Slice(0, inner, 2), :], LIMIT)
            u = jnp.clip(gu_ref[pl.Slice(1, inner, 2), :], -LIMIT, LIMIT)
            glu = g * jax.nn.sigmoid(g * ALPHA)
            gated_ref[:, k * 128:(k + 1) * 128] = ((u + 1.0) * glu).astype(bf16)


def _down_body(te_ref, gated_ref, wd_ref, bd_ref, ad_ref, bdn_ref, y_ref,
               *, num_experts):
    i = pl.program_id(0)
    f32 = jnp.float32
    bf16 = jnp.bfloat16

    @pl.when(te_ref[i] < num_experts)
    def _():
        gated = gated_ref[...]
        y = lax.dot_general(gated, wd_ref[0], (((0,), (0,)), ((), ())),
                            preferred_element_type=f32) + bd_ref[0]
        q = lax.dot_general(gated, ad_ref[0], (((0,), (0,)), ((), ())),
                            preferred_element_type=f32).astype(bf16)
        y = y + SCALING * jnp.dot(q, bdn_ref[0], preferred_element_type=f32)
        y_ref[...] = y


def _combine_body(z0_ref, z1_ref, w0_ref, w1_ref, o_ref):
    o_ref[...] = w0_ref[...] * z0_ref[...] + w1_ref[...] * z1_ref[...]


def kernel(hidden_states, routing_weights, gate_up_proj, gate_up_proj_bias,
           down_proj, down_proj_bias, lora_A_gate_up, lora_B_gate_up,
           lora_A_down, lora_B_down, router_indices):
    batch, seq, hd = hidden_states.shape
    num_experts, _, two_i = gate_up_proj.shape
    inner = two_i // 2
    rank = lora_A_gate_up.shape[-1]
    tokens = batch * seq
    topk = router_indices.shape[1]
    pairs = tokens * topk
    ntiles = pairs // TM + num_experts
    cap = ntiles * TM

    x = hidden_states.reshape(tokens, hd)
    f32 = jnp.float32
    bf16 = jnp.bfloat16

    e_pair = router_indices.reshape(-1).astype(jnp.int32)
    order = jnp.argsort(e_pair, stable=True).astype(jnp.int32)
    sorted_e = e_pair[order]
    eids = jnp.arange(num_experts, dtype=jnp.int32)
    n_e = jnp.sum(e_pair[None, :] == eids[:, None], axis=1).astype(jnp.int32)
    start_e = jnp.concatenate([jnp.zeros((1,), jnp.int32), jnp.cumsum(n_e)[:-1]])
    ntiles_e = (n_e + TM - 1) // TM
    cumtiles = jnp.cumsum(ntiles_e)
    padded_start_e = TM * jnp.concatenate(
        [jnp.zeros((1,), jnp.int32), cumtiles[:-1]])
    q = jnp.arange(cap, dtype=jnp.int32)
    g_q = jnp.searchsorted(TM * cumtiles, q, side="right").astype(jnp.int32)
    g_qc = jnp.minimum(g_q, num_experts - 1)
    off_q = q - padded_start_e[g_qc]
    i_q = jnp.clip(start_e[g_qc] + off_q, 0, pairs - 1)
    valid_q = (g_q < num_experts) & (off_q < n_e[g_qc])
    p_q = order[i_q]
    row_token = jnp.where(valid_q, p_q // topk, 0)
    dest = jnp.where(valid_q, (p_q % topk) * tokens + p_q // topk, pairs)
    tile_expert = jnp.searchsorted(
        cumtiles, jnp.arange(ntiles, dtype=jnp.int32), side="right"
    ).astype(jnp.int32)
    w_pair = routing_weights[
        jnp.arange(pairs, dtype=jnp.int32) // topk, e_pair].reshape(tokens, topk)

    wgu = gate_up_proj.astype(bf16)
    bgu_b = gate_up_proj_bias[:, :, None]
    agu = lora_A_gate_up.astype(bf16)
    bgu_l = lora_B_gate_up.astype(bf16)
    wd = down_proj.astype(bf16)
    bd = down_proj_bias[:, None, :]
    ad = lora_A_down.astype(bf16)
    bdn = lora_B_down.astype(bf16)

    x_pad = _sc_gather_rows(x, row_token).astype(bf16)

    def emap(e3):
        return lambda i, te: (jnp.minimum(te[i], num_experts - 1),) + (0,) * e3

    gu_grid = pltpu.PrefetchScalarGridSpec(
        num_scalar_prefetch=1,
        grid=(ntiles,),
        in_specs=[
            pl.BlockSpec((TM, hd), lambda i, te: (i, 0)),
            pl.BlockSpec((1, hd, two_i), emap(2)),
            pl.BlockSpec((1, two_i, 1), emap(2)),
            pl.BlockSpec((1, hd, rank), emap(2)),
            pl.BlockSpec((1, rank, two_i), emap(2)),
        ],
        out_specs=pl.BlockSpec((inner, TM), lambda i, te: (0, i)),
        scratch_shapes=[pltpu.VMEM((two_i, 128), jnp.float32)],
    )
    gated = pl.pallas_call(
        functools.partial(_gateup_body, num_experts=num_experts),
        grid_spec=gu_grid,
        out_shape=jax.ShapeDtypeStruct((inner, cap), bf16),
    )(tile_expert, x_pad, wgu, bgu_b, agu, bgu_l)

    dn_grid = pltpu.PrefetchScalarGridSpec(
        num_scalar_prefetch=1,
        grid=(ntiles,),
        in_specs=[
            pl.BlockSpec((inner, TM), lambda i, te: (0, i)),
            pl.BlockSpec((1, inner, hd), emap(2)),
            pl.BlockSpec((1, 1, hd), emap(2)),
            pl.BlockSpec((1, inner, rank), emap(2)),
            pl.BlockSpec((1, rank, hd), emap(2)),
        ],
        out_specs=pl.BlockSpec((TM, hd), lambda i, te: (i, 0)),
    )
    y_pad = pl.pallas_call(
        functools.partial(_down_body, num_experts=num_experts),
        grid_spec=dn_grid,
        out_shape=jax.ShapeDtypeStruct((cap, hd), f32),
    )(tile_expert, gated, wd, bd, ad, bdn)

    z = _sc_scatter_rows(y_pad, dest, pairs + 8)

    tm2 = min(512, tokens)
    nt2 = tokens // tm2
    out = pl.pallas_call(
        _combine_body,
        grid=(nt2,),
        in_specs=[
            pl.BlockSpec((tm2, hd), lambda i: (i, 0)),
            pl.BlockSpec((tm2, hd), lambda i: (i + nt2, 0)),
            pl.BlockSpec((tm2, 1), lambda i: (i, 0)),
            pl.BlockSpec((tm2, 1), lambda i: (i, 0)),
        ],
        out_specs=pl.BlockSpec((tm2, hd), lambda i: (i, 0)),
        out_shape=jax.ShapeDtypeStruct((tokens, hd), f32),
    )(z, z, w_pair[:, 0:1], w_pair[:, 1:2])

    return out.reshape(batch, seq, hd)


# speedup vs baseline: 1.1013x; 1.1013x over previous
import functools

import jax
import jax.numpy as jnp
from jax import lax
from jax.experimental import pallas as pl
from jax.experimental.pallas import tpu as pltpu
from jax.experimental.pallas import tpu_sc as plsc

SCALING = 32.0 / 16.0
ALPHA = 1.702
LIMIT = 7.0

TM = 256
SC_CHUNK = 64


def _sc_gather_rows(table, idx):
    n_rows = idx.shape[0]
    width = table.shape[1]
    info = plsc.get_sparse_core_info()
    nc, ns = info.num_cores, info.num_subcores
    nw = nc * ns
    rows_per_w = n_rows // nw
    assert rows_per_w * nw == n_rows
    chunk = next(c for c in (64, 48, 32, 16, 8) if rows_per_w % c == 0)

    mesh = plsc.VectorSubcoreMesh(core_axis_name="c", subcore_axis_name="s")

    @functools.partial(
        pl.kernel, mesh=mesh,
        out_type=jax.ShapeDtypeStruct((n_rows, width), jnp.float32),
        scratch_types=[
            pltpu.VMEM((chunk,), jnp.int32),
            pltpu.VMEM((chunk, width), jnp.float32),
            pltpu.SemaphoreType.DMA,
        ],
    )
    def gather_k(idx_hbm, table_hbm, out_hbm, idx_v, rows_v, sem):
        wid = lax.axis_index("s") * nc + lax.axis_index("c")
        base = wid * rows_per_w
        for c in range(rows_per_w // chunk):
            off = base + c * chunk
            pltpu.sync_copy(idx_hbm.at[pl.ds(off, chunk)], idx_v)
            pltpu.async_copy(table_hbm.at[idx_v], rows_v, sem).wait()
            pltpu.sync_copy(rows_v, out_hbm.at[pl.ds(off, chunk)])

    return gather_k(idx, table)


def _sc_scatter_rows(table, idx, out_rows):
    n_rows, width = table.shape
    info = plsc.get_sparse_core_info()
    nc, ns = info.num_cores, info.num_subcores
    nw = nc * ns
    rows_per_w = n_rows // nw
    assert rows_per_w * nw == n_rows
    chunk = next(c for c in (64, 48, 32, 16, 8) if rows_per_w % c == 0)

    mesh = plsc.VectorSubcoreMesh(core_axis_name="c", subcore_axis_name="s")

    @functools.partial(
        pl.kernel, mesh=mesh,
        out_type=jax.ShapeDtypeStruct((out_rows, width), jnp.float32),
        scratch_types=[
            pltpu.VMEM((chunk,), jnp.int32),
            pltpu.VMEM((chunk, width), jnp.float32),
            pltpu.SemaphoreType.DMA,
        ],
    )
    def scatter_k(idx_hbm, table_hbm, out_hbm, idx_v, rows_v, sem):
        wid = lax.axis_index("s") * nc + lax.axis_index("c")
        base = wid * rows_per_w
        for c in range(rows_per_w // chunk):
            off = base + c * chunk
            pltpu.sync_copy(idx_hbm.at[pl.ds(off, chunk)], idx_v)
            pltpu.sync_copy(table_hbm.at[pl.ds(off, chunk)], rows_v)
            pltpu.async_copy(rows_v, out_hbm.at[idx_v], sem).wait()

    return scatter_k(idx, table)


def _gateup_body(te_ref, x_ref, wgu_ref, bgu_b_ref, agu_ref, bgu_l_ref,
                 gated_ref, gu_ref, *, num_experts):
    i = pl.program_id(0)
    f32 = jnp.float32
    bf16 = jnp.bfloat16
    inner = gated_ref.shape[0]

    @pl.when(te_ref[i] < num_experts)
    def _():
        x = x_ref[...]
        p = jnp.dot(x, agu_ref[0], preferred_element_type=f32).astype(bf16)
        gu_t = lax.dot_general(wgu_ref[0], x, (((0,), (1,)), ((), ())),
                               preferred_element_type=f32)
        dlt = lax.dot_general(bgu_l_ref[0], p, (((0,), (1,)), ((), ())),
                              preferred_element_type=f32)
        gu_t = gu_t + bgu_b_ref[0] + SCALING * dlt
        for k in range(gu_t.shape[1] // 128):
            gu_ref[...] = gu_t[:, k * 128:(k + 1) * 128]
            g = jnp.minimum(gu_ref[pl.Slice(0, inner, 2), :], LIMIT)
            u = jnp.clip(gu_ref[pl.Slice(1, inner, 2), :], -LIMIT, LIMIT)
            glu = g * jax.nn.sigmoid(g * ALPHA)
            gated_ref[:, k * 128:(k + 1) * 128] = ((u + 1.0) * glu).astype(bf16)


def _down_body(te_ref, gated_ref, wd_ref, bd_ref, ad_ref, bdn_ref, y_ref,
               *, num_experts):
    i = pl.program_id(0)
    f32 = jnp.float32
    bf16 = jnp.bfloat16

    @pl.when(te_ref[i] < num_experts)
    def _():
        gated = gated_ref[...]
        y = lax.dot_general(gated, wd_ref[0], (((0,), (0,)), ((), ())),
                            preferred_element_type=f32) + bd_ref[0]
        q = lax.dot_general(gated, ad_ref[0], (((0,), (0,)), ((), ())),
                            preferred_element_type=f32).astype(bf16)
        y = y + SCALING * jnp.dot(q, bdn_ref[0], preferred_element_type=f32)
        y_ref[...] = y


def _combine_body(z0_ref, z1_ref, w0_ref, w1_ref, o_ref):
    o_ref[...] = w0_ref[...] * z0_ref[...] + w1_ref[...] * z1_ref[...]


def kernel(hidden_states, routing_weights, gate_up_proj, gate_up_proj_bias,
           down_proj, down_proj_bias, lora_A_gate_up, lora_B_gate_up,
           lora_A_down, lora_B_down, router_indices):
    batch, seq, hd = hidden_states.shape
    num_experts, _, two_i = gate_up_proj.shape
    inner = two_i // 2
    rank = lora_A_gate_up.shape[-1]
    tokens = batch * seq
    topk = router_indices.shape[1]
    pairs = tokens * topk
    ntiles = pairs // TM + num_experts
    cap = ntiles * TM

    x = hidden_states.reshape(tokens, hd)
    f32 = jnp.float32
    bf16 = jnp.bfloat16

    e_pair = router_indices.reshape(-1).astype(jnp.int32)
    order = jnp.argsort(e_pair, stable=True).astype(jnp.int32)
    sorted_e = e_pair[order]
    eids = jnp.arange(num_experts, dtype=jnp.int32)
    n_e = jnp.sum(e_pair[None, :] == eids[:, None], axis=1).astype(jnp.int32)
    start_e = jnp.concatenate([jnp.zeros((1,), jnp.int32), jnp.cumsum(n_e)[:-1]])
    ntiles_e = (n_e + TM - 1) // TM
    cumtiles = jnp.cumsum(ntiles_e)
    padded_start_e = TM * jnp.concatenate(
        [jnp.zeros((1,), jnp.int32), cumtiles[:-1]])
    rank_in_group = jnp.arange(pairs, dtype=jnp.int32) - start_e[sorted_e]
    dst = padded_start_e[sorted_e] + rank_in_group
    row_token = jnp.zeros((cap,), jnp.int32).at[dst].set(order // topk)
    pair_pos = jnp.zeros((pairs,), jnp.int32).at[
        (order % topk) * tokens + order // topk].set(dst)
    tile_expert = jnp.searchsorted(
        cumtiles, jnp.arange(ntiles, dtype=jnp.int32), side="right"
    ).astype(jnp.int32)
    w_pair = routing_weights[
        jnp.arange(pairs, dtype=jnp.int32) // topk, e_pair].reshape(tokens, topk)

    wgu = gate_up_proj.astype(bf16)
    bgu_b = gate_up_proj_bias[:, :, None]
    agu = lora_A_gate_up.astype(bf16)
    bgu_l = lora_B_gate_up.astype(bf16)
    wd = down_proj.astype(bf16)
    bd = down_proj_bias[:, None, :]
    ad = lora_A_down.astype(bf16)
    bdn = lora_B_down.astype(bf16)

    x_pad = _sc_gather_rows(x, row_token).astype(bf16)

    def emap(e3):
        return lambda i, te: (jnp.minimum(te[i], num_experts - 1),) + (0,) * e3

    gu_grid = pltpu.PrefetchScalarGridSpec(
        num_scalar_prefetch=1,
        grid=(ntiles,),
        in_specs=[
            pl.BlockSpec((TM, hd), lambda i, te: (i, 0)),
            pl.BlockSpec((1, hd, two_i), emap(2)),
            pl.BlockSpec((1, two_i, 1), emap(2)),
            pl.BlockSpec((1, hd, rank), emap(2)),
            pl.BlockSpec((1, rank, two_i), emap(2)),
        ],
        out_specs=pl.BlockSpec((inner, TM), lambda i, te: (0, i)),
        scratch_shapes=[pltpu.VMEM((two_i, 128), jnp.float32)],
    )
    gated = pl.pallas_call(
        functools.partial(_gateup_body, num_experts=num_experts),
        grid_spec=gu_grid,
        out_shape=jax.ShapeDtypeStruct((inner, cap), bf16),
    )(tile_expert, x_pad, wgu, bgu_b, agu, bgu_l)

    dn_grid = pltpu.PrefetchScalarGridSpec(
        num_scalar_prefetch=1,
        grid=(ntiles,),
        in_specs=[
            pl.BlockSpec((inner, TM), lambda i, te: (0, i)),
            pl.BlockSpec((1, inner, hd), emap(2)),
            pl.BlockSpec((1, 1, hd), emap(2)),
            pl.BlockSpec((1, inner, rank), emap(2)),
            pl.BlockSpec((1, rank, hd), emap(2)),
        ],
        out_specs=pl.BlockSpec((TM, hd), lambda i, te: (i, 0)),
    )
    y_pad = pl.pallas_call(
        functools.partial(_down_body, num_experts=num_experts),
        grid_spec=dn_grid,
        out_shape=jax.ShapeDtypeStruct((cap, hd), f32),
    )(tile_expert, gated, wd, bd, ad, bdn)

    z = _sc_gather_rows(y_pad, pair_pos)

    tm2 = min(512, tokens)
    nt2 = tokens // tm2
    out = pl.pallas_call(
        _combine_body,
        grid=(nt2,),
        in_specs=[
            pl.BlockSpec((tm2, hd), lambda i: (i, 0)),
            pl.BlockSpec((tm2, hd), lambda i: (i + nt2, 0)),
            pl.BlockSpec((tm2, 1), lambda i: (i, 0)),
            pl.BlockSpec((tm2, 1), lambda i: (i, 0)),
        ],
        out_specs=pl.BlockSpec((tm2, hd), lambda i: (i, 0)),
        out_shape=jax.ShapeDtypeStruct((tokens, hd), f32),
    )(z, z, w_pair[:, 0:1], w_pair[:, 1:2])

    return out.reshape(batch, seq, hd)
